# Initial kernel scaffold; baseline (speedup 1.0000x reference)
import jax
import jax.numpy as jnp
from jax.experimental import pallas as pl


def kernel(input_ids, emb, W, b):
    raise NotImplementedError("write your pallas kernel here")


# speedup vs baseline: 1.5183x; 1.5183x over previous
import functools

import jax
import jax.numpy as jnp
from jax import lax
from jax.experimental import pallas as pl
from jax.experimental.pallas import tpu as pltpu
from jax.experimental.pallas import tpu_sc as plsc

D = 1024
V = 64
NC, NS = 2, 16
NW = NC * NS
B = 4 * 2048
BPW = B // NW
CHUNK = 64
NCHUNK = BPW // CHUNK
IDS_ROWS = B // 128


def _table_body(emb_ref, w_ref, b_ref, ids_ref, table_ref, mean_ref):
    t = lax.dot_general(emb_ref[...], w_ref[...], (((1,), (1,)), ((), ())),
                        preferred_element_type=jnp.float32)
    t = t + b_ref[...]
    table_ref[...] = t
    m = jnp.mean(t, axis=1, keepdims=True)
    ids = ids_ref[...]
    acc = jnp.zeros((IDS_ROWS, 128), jnp.float32)
    for v in range(V):
        acc = acc + jnp.where(ids == v, m[v, 0], 0.0)
    mean_ref[...] = acc


_table_call = pl.pallas_call(
    _table_body,
    out_shape=[
        jax.ShapeDtypeStruct((V, D), jnp.float32),
        jax.ShapeDtypeStruct((IDS_ROWS, 128), jnp.float32),
    ],
)


_sc_mesh = plsc.VectorSubcoreMesh(
    core_axis_name="c", subcore_axis_name="s", num_cores=NC, num_subcores=NS)


@functools.partial(
    pl.kernel,
    out_type=jax.ShapeDtypeStruct((B, D), jnp.float32),
    mesh=_sc_mesh,
    scratch_types=[
        pltpu.VMEM((BPW,), jnp.int32),
        pltpu.VMEM((CHUNK, D), jnp.float32),
        pltpu.SemaphoreType.DMA,
    ],
)
def _sc_gather(table_hbm, ids_hbm, y_hbm, idx_v, rows_v, sem):
    wid = lax.axis_index("s") * NC + lax.axis_index("c")
    base = wid * BPW
    pltpu.sync_copy(ids_hbm.at[pl.ds(base, BPW)], idx_v)
    for c in range(NCHUNK):
        pltpu.async_copy(
            table_hbm.at[idx_v.at[pl.ds(c * CHUNK, CHUNK)]], rows_v, sem
        ).wait()
        pltpu.sync_copy(rows_v, y_hbm.at[pl.ds(base + c * CHUNK, CHUNK)])


def kernel(input_ids, emb, W, b):
    bdim, sdim = input_ids.shape
    ids = input_ids.reshape(-1).astype(jnp.int32)
    table, mean2d = _table_call(emb, W, b.reshape(1, D),
                                ids.reshape(IDS_ROWS, 128))
    y_flat = _sc_gather(table, ids)
    return (y_flat.reshape(bdim, sdim, D), mean2d.reshape(bdim, sdim))
